# baseline (speedup 1.0000x reference)
import functools

import jax
import jax.numpy as jnp
from jax import lax
from jax.experimental import pallas as pl
from jax.experimental.pallas import tpu as pltpu
from jax.experimental.pallas import tpu_sc as plsc

N_NODES = 10000
N_EDGES = 320000
DIM = 128

NC = 2
NS = 16
NW = NC * NS
CHUNK = 128
N_CHUNKS = N_EDGES // CHUNK
IB = 16
NCH = 80
NB = NCH // IB
N_CHUNKS_PAD = NW * NCH
N_PAD = 10240
ROWS_PER_TILE = N_PAD // NS
DUMP = 128
DEG_W = 16
IN_COL = 64

_MESH = plsc.VectorSubcoreMesh(core_axis_name="c", subcore_axis_name="s")


def _worker():
    c = lax.axis_index("c")
    s = lax.axis_index("s")
    wid = c * NS + s
    start = wid * NCH
    count = jnp.clip(N_CHUNKS - start, 0, NCH)
    return c, s, start, count


@functools.partial(
    pl.kernel,
    out_type=jax.ShapeDtypeStruct((NC, N_PAD, DIM), jnp.float32),
    mesh=_MESH,
    scratch_types=[
        pltpu.VMEM((IB, CHUNK), jnp.int32),
        pltpu.VMEM((IB, CHUNK), jnp.int32),
        pltpu.VMEM((CHUNK, DIM), jnp.float32),
        pltpu.VMEM((CHUNK, DIM), jnp.float32),
        pltpu.VMEM_SHARED((N_PAD, DIM), jnp.float32),
    ],
)
def _deg_kernel(src_hbm, dst_hbm, onesa_hbm, onesb_hbm, zeros_hbm, deg_hbm,
                idx_s, idx_d, ones_a, ones_b, acc):
    c, s, start, count = _worker()

    r0 = s * ROWS_PER_TILE
    pltpu.sync_copy(zeros_hbm, ones_b)
    for j in range(ROWS_PER_TILE // DUMP):
        pltpu.sync_copy(ones_b, acc.at[pl.ds(r0 + j * DUMP, DUMP)])
    plsc.subcore_barrier()
    pltpu.sync_copy(onesa_hbm, ones_a)
    pltpu.sync_copy(onesb_hbm, ones_b)

    def block_body(b, _):
        bbase = start + b * IB
        pltpu.sync_copy(src_hbm.at[pl.ds(bbase, IB)], idx_s)
        pltpu.sync_copy(dst_hbm.at[pl.ds(bbase, IB)], idx_d)

        def inner(off, _):
            j = b * IB + off
            @pl.when(j < count)
            def _():
                pltpu.sync_copy(ones_a, acc.at[idx_s.at[off]], add=True)
                pltpu.sync_copy(ones_b, acc.at[idx_d.at[off]], add=True)
            return 0
        lax.fori_loop(0, IB, inner, 0)
        return 0
    lax.fori_loop(0, NB, block_body, 0)
    plsc.subcore_barrier()

    for j in range(ROWS_PER_TILE // DUMP):
        pltpu.sync_copy(acc.at[pl.ds(r0 + j * DUMP, DUMP)], ones_b)
        pltpu.sync_copy(ones_b, deg_hbm.at[c, pl.ds(r0 + j * DUMP, DUMP)])


@functools.partial(
    pl.kernel,
    out_type=jax.ShapeDtypeStruct((NC, N_PAD, DIM), jnp.float32),
    mesh=_MESH,
    scratch_types=[
        pltpu.VMEM((IB, CHUNK), jnp.int32),
        pltpu.VMEM((IB, CHUNK), jnp.int32),
        pltpu.VMEM((2, CHUNK, DIM), jnp.float32),
        pltpu.VMEM_SHARED((N_PAD, DIM), jnp.float32),
        pltpu.SemaphoreType.DMA,
        pltpu.SemaphoreType.DMA,
    ],
)
def _gs_kernel(y_hbm, src_hbm, dst_hbm, zeros_hbm, out_hbm,
               idx_s, idx_d, rows, acc, sa, sb):
    c, s, start, count = _worker()
    stage = rows.at[0]
    pltpu.sync_copy(zeros_hbm, stage)

    r0 = s * ROWS_PER_TILE
    for j in range(ROWS_PER_TILE // DUMP):
        pltpu.sync_copy(stage, acc.at[pl.ds(r0 + j * DUMP, DUMP)])
    plsc.subcore_barrier()

    def block_body(b, _):
        bbase = start + b * IB
        pltpu.sync_copy(src_hbm.at[pl.ds(bbase, IB)], idx_s)
        pltpu.sync_copy(dst_hbm.at[pl.ds(bbase, IB)], idx_d)

        pltpu.async_copy(y_hbm.at[idx_s.at[0]], rows.at[0], sa)

        def pair(p, _):
            ja = b * IB + 2 * p
            jb = ja + 1
            oa = 2 * p
            pltpu.async_copy(y_hbm.at[idx_s.at[oa + 1]], rows.at[1], sb)
            pltpu.make_async_copy(y_hbm.at[idx_s.at[oa]], rows.at[0], sa).wait()
            @pl.when(ja < count)
            def _():
                pltpu.sync_copy(rows.at[0], acc.at[idx_d.at[oa]], add=True)
            @pl.when(oa + 2 < IB)
            def _():
                pltpu.async_copy(y_hbm.at[idx_s.at[oa + 2]], rows.at[0], sa)
            pltpu.make_async_copy(y_hbm.at[idx_s.at[oa + 1]], rows.at[1], sb).wait()
            @pl.when(jb < count)
            def _():
                pltpu.sync_copy(rows.at[1], acc.at[idx_d.at[oa + 1]], add=True)
            return 0
        lax.fori_loop(0, IB // 2, pair, 0)
        return 0
    lax.fori_loop(0, NB, block_body, 0)
    plsc.subcore_barrier()

    for j in range(ROWS_PER_TILE // DUMP):
        pltpu.sync_copy(acc.at[pl.ds(r0 + j * DUMP, DUMP)], stage)
        pltpu.sync_copy(stage, out_hbm.at[c, pl.ds(r0 + j * DUMP, DUMP)])


def _prep_body(deg_ref, x_ref, y_ref, oinv_ref, iinv_ref):
    od = deg_ref[0, :N_NODES, 0:1] + deg_ref[1, :N_NODES, 0:1]
    idg = (deg_ref[0, :N_NODES, IN_COL:IN_COL + 1]
           + deg_ref[1, :N_NODES, IN_COL:IN_COL + 1])
    oinv = lax.rsqrt(jnp.maximum(od, 1.0))
    iinv = lax.rsqrt(jnp.maximum(idg, 1.0))
    oinv_ref[...] = jnp.broadcast_to(oinv, (N_NODES, DEG_W))
    iinv_ref[...] = jnp.broadcast_to(iinv, (N_NODES, DEG_W))
    y_ref[...] = x_ref[...] * oinv


def _prep_call(deg, x):
    return pl.pallas_call(
        _prep_body,
        out_shape=(
            jax.ShapeDtypeStruct((N_NODES, DIM), jnp.float32),
            jax.ShapeDtypeStruct((N_NODES, DEG_W), jnp.float32),
            jax.ShapeDtypeStruct((N_NODES, DEG_W), jnp.float32),
        ),
    )(deg, x)


def _mid_body(acc_ref, iinv_ref, oinv_ref, w_ref, b_ref, y2_ref):
    a = acc_ref[0, :N_NODES] + acc_ref[1, :N_NODES]
    agg = a * iinv_ref[...][:, :1]
    o = jnp.dot(agg, w_ref[...], preferred_element_type=jnp.float32)
    h = jnp.maximum(o + b_ref[...], 0.0)
    y2_ref[...] = h * oinv_ref[...][:, :1]


def _mid_call(acc, iinv, oinv, W, b):
    return pl.pallas_call(
        _mid_body,
        out_shape=jax.ShapeDtypeStruct((N_NODES, DIM), jnp.float32),
    )(acc, iinv, oinv, W, b.reshape(1, DIM))


def _final_body(acc_ref, iinv_ref, w_ref, b_ref, out_ref):
    a = acc_ref[0, :N_NODES] + acc_ref[1, :N_NODES]
    agg = a * iinv_ref[...][:, :1]
    o = jnp.dot(agg, w_ref[...], preferred_element_type=jnp.float32)
    out_ref[...] = jnp.maximum(o + b_ref[...], 0.0)


def _final_call(acc, iinv, W, b):
    return pl.pallas_call(
        _final_body,
        out_shape=jax.ShapeDtypeStruct((N_NODES, DIM), jnp.float32),
    )(acc, iinv, W, b.reshape(1, DIM))


def kernel(node_feature, edge_index, W1, b1, W2, b2):
    ei = edge_index.astype(jnp.int32)
    pad = ((0, N_CHUNKS_PAD - N_CHUNKS), (0, 0))
    src = jnp.pad(ei[0].reshape(N_CHUNKS, CHUNK), pad)
    dst = jnp.pad(ei[1].reshape(N_CHUNKS, CHUNK), pad)
    col = jnp.arange(DIM)
    ones_a = jnp.broadcast_to(
        jnp.where(col < 16, 1.0, 0.0), (CHUNK, DIM)).astype(jnp.float32)
    ones_b = jnp.broadcast_to(
        ((col >= IN_COL) & (col < IN_COL + 16)).astype(jnp.float32),
        (CHUNK, DIM))
    zeros_f32 = jnp.zeros((DUMP, DIM), jnp.float32)
    deg = _deg_kernel(src, dst, ones_a, ones_b, zeros_f32)
    y1, oinv, iinv = _prep_call(deg, node_feature)
    acc1 = _gs_kernel(y1, src, dst, zeros_f32)
    y2 = _mid_call(acc1, iinv, oinv, W1, b1)
    acc2 = _gs_kernel(y2, src, dst, zeros_f32)
    return _final_call(acc2, iinv, W2, b2)


# speedup vs baseline: 2.5444x; 2.5444x over previous
import functools

import jax
import jax.numpy as jnp
from jax import lax
from jax.experimental import pallas as pl
from jax.experimental.pallas import tpu as pltpu
from jax.experimental.pallas import tpu_sc as plsc

N_NODES = 10000
N_EDGES = 320000
DIM = 128

NC = 2
NS = 16
NW = NC * NS
CHUNK = 128
N_CHUNKS = N_EDGES // CHUNK
IB = 16
NCH = 80
NB = NCH // IB
N_CHUNKS_PAD = NW * NCH
N_PAD = 10240
ROWS_PER_TILE = N_PAD // NS
DUMP = 128
DEG_W = 16
IN_COL = 64

_MESH = plsc.VectorSubcoreMesh(core_axis_name="c", subcore_axis_name="s")


def _worker():
    c = lax.axis_index("c")
    s = lax.axis_index("s")
    wid = c * NS + s
    start = wid * NCH
    count = jnp.clip(N_CHUNKS - start, 0, NCH)
    return c, s, start, count


@functools.partial(
    pl.kernel,
    out_type=jax.ShapeDtypeStruct((NC, N_PAD, DIM), jnp.float32),
    mesh=_MESH,
    scratch_types=[
        pltpu.VMEM((IB, CHUNK), jnp.int32),
        pltpu.VMEM((IB, CHUNK), jnp.int32),
        pltpu.VMEM((CHUNK, DIM), jnp.float32),
        pltpu.VMEM((CHUNK, DIM), jnp.float32),
        pltpu.VMEM_SHARED((N_PAD, DIM), jnp.float32),
    ],
)
def _deg_kernel(src_hbm, dst_hbm, onesa_hbm, onesb_hbm, zeros_hbm, deg_hbm,
                idx_s, idx_d, ones_a, ones_b, acc):
    c, s, start, count = _worker()

    r0 = s * ROWS_PER_TILE
    pltpu.sync_copy(zeros_hbm, ones_b)
    for j in range(ROWS_PER_TILE // DUMP):
        pltpu.sync_copy(ones_b, acc.at[pl.ds(r0 + j * DUMP, DUMP)])
    plsc.subcore_barrier()
    pltpu.sync_copy(onesa_hbm, ones_a)
    pltpu.sync_copy(onesb_hbm, ones_b)

    def block_body(b, _):
        bbase = start + b * IB
        pltpu.sync_copy(src_hbm.at[pl.ds(bbase, IB)], idx_s)
        pltpu.sync_copy(dst_hbm.at[pl.ds(bbase, IB)], idx_d)

        def inner(off, _):
            j = b * IB + off
            @pl.when(j < count)
            def _():
                pltpu.sync_copy(ones_a, acc.at[idx_s.at[off]], add=True)
                pltpu.sync_copy(ones_b, acc.at[idx_d.at[off]], add=True)
            return 0
        lax.fori_loop(0, IB, inner, 0)
        return 0
    lax.fori_loop(0, NB, block_body, 0)
    plsc.subcore_barrier()

    for j in range(ROWS_PER_TILE // DUMP):
        pltpu.sync_copy(acc.at[pl.ds(r0 + j * DUMP, DUMP)], ones_b)
        pltpu.sync_copy(ones_b, deg_hbm.at[c, pl.ds(r0 + j * DUMP, DUMP)])


@functools.partial(
    pl.kernel,
    out_type=jax.ShapeDtypeStruct((NC, N_PAD, DIM), jnp.float32),
    mesh=_MESH,
    scratch_types=[
        pltpu.VMEM((IB, CHUNK), jnp.int32),
        pltpu.VMEM((IB, CHUNK), jnp.int32),
        pltpu.VMEM((2, CHUNK, DIM), jnp.float32),
        pltpu.VMEM_SHARED((N_PAD, DIM), jnp.float32),
        pltpu.SemaphoreType.DMA,
        pltpu.SemaphoreType.DMA,
    ],
)
def _gs_kernel(y_hbm, src_hbm, dst_hbm, zeros_hbm, out_hbm,
               idx_s, idx_d, rows, acc, sa, sb):
    c, s, start, count = _worker()
    stage = rows.at[0]
    pltpu.sync_copy(zeros_hbm, stage)

    r0 = s * ROWS_PER_TILE
    for j in range(ROWS_PER_TILE // DUMP):
        pltpu.sync_copy(stage, acc.at[pl.ds(r0 + j * DUMP, DUMP)])
    plsc.subcore_barrier()

    def block_body(b, _):
        bbase = start + b * IB
        pltpu.sync_copy(src_hbm.at[pl.ds(bbase, IB)], idx_s)
        pltpu.sync_copy(dst_hbm.at[pl.ds(bbase, IB)], idx_d)

        pltpu.async_copy(y_hbm.at[idx_s.at[0]], rows.at[0], sa)

        def pair(p, _):
            ja = b * IB + 2 * p
            jb = ja + 1
            oa = 2 * p
            pltpu.async_copy(y_hbm.at[idx_s.at[oa + 1]], rows.at[1], sb)
            pltpu.make_async_copy(y_hbm.at[idx_s.at[oa]], rows.at[0], sa).wait()
            @pl.when(ja < count)
            def _():
                pltpu.sync_copy(rows.at[0], acc.at[idx_d.at[oa]], add=True)
            @pl.when(oa + 2 < IB)
            def _():
                pltpu.async_copy(y_hbm.at[idx_s.at[oa + 2]], rows.at[0], sa)
            pltpu.make_async_copy(y_hbm.at[idx_s.at[oa + 1]], rows.at[1], sb).wait()
            @pl.when(jb < count)
            def _():
                pltpu.sync_copy(rows.at[1], acc.at[idx_d.at[oa + 1]], add=True)
            return 0
        lax.fori_loop(0, IB // 2, pair, 0)
        return 0
    lax.fori_loop(0, NB, block_body, 0)
    plsc.subcore_barrier()

    for j in range(ROWS_PER_TILE // DUMP):
        pltpu.sync_copy(acc.at[pl.ds(r0 + j * DUMP, DUMP)], stage)
        pltpu.sync_copy(stage, out_hbm.at[c, pl.ds(r0 + j * DUMP, DUMP)])


def _prep_body(deg_ref, x_ref, y_ref, oinv_ref, iinv_ref):
    od = deg_ref[0, :N_NODES, 0:1] + deg_ref[1, :N_NODES, 0:1]
    idg = (deg_ref[0, :N_NODES, IN_COL:IN_COL + 1]
           + deg_ref[1, :N_NODES, IN_COL:IN_COL + 1])
    oinv = lax.rsqrt(jnp.maximum(od, 1.0))
    iinv = lax.rsqrt(jnp.maximum(idg, 1.0))
    oinv_ref[...] = jnp.broadcast_to(oinv, (N_NODES, DEG_W))
    iinv_ref[...] = jnp.broadcast_to(iinv, (N_NODES, DEG_W))
    y_ref[...] = x_ref[...] * oinv


def _prep_call(deg, x):
    return pl.pallas_call(
        _prep_body,
        out_shape=(
            jax.ShapeDtypeStruct((N_NODES, DIM), jnp.float32),
            jax.ShapeDtypeStruct((N_NODES, DEG_W), jnp.float32),
            jax.ShapeDtypeStruct((N_NODES, DEG_W), jnp.float32),
        ),
    )(deg, x)


def _mid_body(acc_ref, iinv_ref, oinv_ref, w_ref, b_ref, y2_ref):
    a = acc_ref[0, :N_NODES] + acc_ref[1, :N_NODES]
    agg = a * iinv_ref[...][:, :1]
    o = jnp.dot(agg, w_ref[...], preferred_element_type=jnp.float32)
    h = jnp.maximum(o + b_ref[...], 0.0)
    y2_ref[...] = h * oinv_ref[...][:, :1]


def _mid_call(acc, iinv, oinv, W, b):
    return pl.pallas_call(
        _mid_body,
        out_shape=jax.ShapeDtypeStruct((N_NODES, DIM), jnp.float32),
    )(acc, iinv, oinv, W, b.reshape(1, DIM))


def _final_body(acc_ref, iinv_ref, w_ref, b_ref, out_ref):
    a = acc_ref[0, :N_NODES] + acc_ref[1, :N_NODES]
    agg = a * iinv_ref[...][:, :1]
    o = jnp.dot(agg, w_ref[...], preferred_element_type=jnp.float32)
    out_ref[...] = jnp.maximum(o + b_ref[...], 0.0)


def _final_call(acc, iinv, W, b):
    return pl.pallas_call(
        _final_body,
        out_shape=jax.ShapeDtypeStruct((N_NODES, DIM), jnp.float32),
    )(acc, iinv, W, b.reshape(1, DIM))


def kernel(node_feature, edge_index, W1, b1, W2, b2):
    ei = edge_index.astype(jnp.int32)
    pad = ((0, N_CHUNKS_PAD - N_CHUNKS), (0, 0))
    pad_row = jnp.broadcast_to((jnp.arange(CHUNK, dtype=jnp.int32) * 64)
                               % N_NODES, (N_CHUNKS_PAD - N_CHUNKS, CHUNK))
    src = jnp.concatenate([ei[0].reshape(N_CHUNKS, CHUNK), pad_row])
    dst = jnp.pad(ei[1].reshape(N_CHUNKS, CHUNK), pad)
    col = jnp.arange(DIM)
    ones_a = jnp.broadcast_to(
        jnp.where(col < 16, 1.0, 0.0), (CHUNK, DIM)).astype(jnp.float32)
    ones_b = jnp.broadcast_to(
        ((col >= IN_COL) & (col < IN_COL + 16)).astype(jnp.float32),
        (CHUNK, DIM))
    zeros_f32 = jnp.zeros((DUMP, DIM), jnp.float32)
    deg = _deg_kernel(src, dst, ones_a, ones_b, zeros_f32)
    y1, oinv, iinv = _prep_call(deg, node_feature)
    acc1 = _gs_kernel(y1, src, dst, zeros_f32)
    y2 = _mid_call(acc1, iinv, oinv, W1, b1)
    acc2 = _gs_kernel(y2, src, dst, zeros_f32)
    return _final_call(acc2, iinv, W2, b2)
